# baseline (speedup 1.0000x reference)
import functools

import jax
import jax.numpy as jnp
from jax import lax
from jax.experimental import pallas as pl
from jax.experimental.pallas import tpu as pltpu
from jax.experimental.pallas import tpu_sc as plsc

_EPS = 1e-6
_L = 16


def _rsqrt_newton(x):
    i = plsc.bitcast(x, jnp.int32)
    i = jnp.full(x.shape, 0x5F3759DF, jnp.int32) - lax.shift_right_logical(i, 1)
    y = plsc.bitcast(i, jnp.float32)
    for _ in range(3):
        y = y * (1.5 - 0.5 * x * y * y)
    return y


@functools.lru_cache(maxsize=None)
def _build(total, M, B):
    mesh = plsc.VectorSubcoreMesh(core_axis_name="c", subcore_axis_name="s")
    NC, NS = mesh.num_cores, mesh.num_subcores
    NW = NC * NS
    CW = total // NW
    SUB = 256
    NT = CW // SUB
    KV = SUB // _L
    SUBN = 128
    NTN = CW // SUBN
    KVN = SUBN // _L
    NB = 4
    assert total == NW * NT * SUB and CW % SUBN == 0

    def seg_cols(off_vec, i):
        lo = off_vec[i]
        hi = jnp.int32(total) if i == B - 1 else off_vec[i + 1]
        return lo, hi

    @functools.partial(
        pl.kernel,
        out_type=jax.ShapeDtypeStruct((NW * 2 * _L,), jnp.float32),
        mesh=mesh,
        compiler_params=pltpu.CompilerParams(needs_layout_passes=False),
        scratch_types=[
            pltpu.VMEM((M, SUB), jnp.float32),
            pltpu.VMEM((M, SUB), jnp.float32),
            pltpu.VMEM((CW,), jnp.float32),
            pltpu.VMEM((CW,), jnp.float32),
            pltpu.VMEM((_L,), jnp.int32),
            pltpu.VMEM((2 * _L,), jnp.float32),
            pltpu.SemaphoreType.DMA,
            pltpu.SemaphoreType.DMA,
        ],
    )
    def stats_k(vt_hbm, offsets_hbm, part_hbm,
                chunk0, chunk1, colsum, colsq, offs, stat_v, sem0, sem1):
        zeros = jnp.zeros((_L,), jnp.float32)
        lane_iota = lax.iota(jnp.int32, _L)
        wid = lax.axis_index("c") * NS + lax.axis_index("s")
        bufs = [chunk0, chunk1]
        sems = [sem0, sem1]

        def start_load(t):
            cb = wid * CW + t * SUB
            return pltpu.async_copy(
                vt_hbm.at[:, pl.ds(cb, SUB)], bufs[t % 2], sems[t % 2])

        loads = {0: start_load(0)}
        pltpu.sync_copy(offsets_hbm.at[pl.ds(0, _L)], offs)
        off_vec = offs[...]
        sums_vec = zeros
        sq_vec = zeros
        for t in range(NT):
            if t + 1 < NT:
                loads[t + 1] = start_load(t + 1)
            loads[t].wait()
            cbase = wid * CW + t * SUB
            chunk = bufs[t % 2]

            def kbody(k, _):
                def mbody(m, carry):
                    s0, q0, s1, q1 = carry
                    v0 = chunk[m, pl.ds(k * 2 * _L, _L)]
                    v1 = chunk[m, pl.ds(k * 2 * _L + _L, _L)]
                    return s0 + v0, q0 + v0 * v0, s1 + v1, q1 + v1 * v1

                s0, q0, s1, q1 = plsc.parallel_loop(
                    0, M, unroll=8,
                    carry=(zeros, zeros, zeros, zeros))(mbody)
                cb = t * SUB + k * 2 * _L
                colsum[pl.ds(cb, _L)] = s0
                colsq[pl.ds(cb, _L)] = q0
                colsum[pl.ds(cb + _L, _L)] = s1
                colsq[pl.ds(cb + _L, _L)] = q1
                return 0

            lax.fori_loop(0, KV // 2, kbody, 0)

        wbase = wid * CW
        for i in range(B):
            lo, hi = seg_cols(off_vec, i)
            ra = jnp.clip(lo - wbase, 0, CW)
            rb = jnp.clip(hi - wbase, 0, CW)

            def sbody(kk, carry):
                s, q = carry
                g = kk * _L + lane_iota
                msk = (g >= ra) & (g < rb)
                s = s + jnp.where(msk, colsum[pl.ds(kk * _L, _L)], 0.0)
                q = q + jnp.where(msk, colsq[pl.ds(kk * _L, _L)], 0.0)
                return s, q

            s, q = lax.fori_loop(
                lax.div(ra, _L), lax.div(rb + (_L - 1), _L),
                sbody, (zeros, zeros))
            lane = lane_iota == i
            sums_vec = jnp.where(lane, sums_vec + jnp.sum(s), sums_vec)
            sq_vec = jnp.where(lane, sq_vec + jnp.sum(q), sq_vec)

        stat_v[pl.ds(0, _L)] = sums_vec
        stat_v[pl.ds(_L, _L)] = sq_vec
        pltpu.sync_copy(stat_v, part_hbm.at[pl.ds(wid * 2 * _L, 2 * _L)])

    @functools.partial(
        pl.kernel,
        out_type=jax.ShapeDtypeStruct((M, total), jnp.float32),
        mesh=mesh,
        compiler_params=pltpu.CompilerParams(needs_layout_passes=False),
        scratch_types=(
            [pltpu.VMEM((M, SUBN), jnp.float32)] * NB
            + [pltpu.VMEM((_L,), jnp.int32),
               pltpu.VMEM((NW * 2 * _L,), jnp.float32)]
            + [pltpu.SemaphoreType.DMA] * (2 * NB)
        ),
    )
    def norm_k(vt_hbm, offsets_hbm, part_hbm, out_hbm, *refs):
        bufs = list(refs[:NB])
        offs, part_v = refs[NB], refs[NB + 1]
        lsems = list(refs[NB + 2:NB + 2 + NB])
        ssems = list(refs[NB + 2 + NB:])
        zeros = jnp.zeros((_L,), jnp.float32)
        lane_iota = lax.iota(jnp.int32, _L)
        wid = lax.axis_index("c") * NS + lax.axis_index("s")

        def start_load(t):
            cb = wid * CW + t * SUBN
            return pltpu.async_copy(
                vt_hbm.at[:, pl.ds(cb, SUBN)], bufs[t % NB], lsems[t % NB])

        def start_store(t):
            cb = wid * CW + t * SUBN
            return pltpu.async_copy(
                bufs[t % NB], out_hbm.at[:, pl.ds(cb, SUBN)], ssems[t % NB])

        loads = {0: start_load(0), 1: start_load(1)}
        pltpu.sync_copy(offsets_hbm.at[pl.ds(0, _L)], offs)
        pltpu.sync_copy(part_hbm, part_v)
        off_vec = offs[...]

        sums = zeros
        sqs = zeros
        for w in range(NW):
            sums = sums + part_v[pl.ds(w * 2 * _L, _L)]
            sqs = sqs + part_v[pl.ds(w * 2 * _L + _L, _L)]

        off_hi = jnp.full((_L,), total, jnp.int32)
        for i in range(B - 1):
            off_hi = jnp.where(lane_iota == i, off_vec[i + 1], off_hi)
        n_elem = (off_hi - off_vec).astype(jnp.float32) * jnp.float32(M)

        mean = sums / n_elem
        var = sqs / n_elem - mean * mean
        rstd = _rsqrt_newton(var + _EPS)

        stores = {}
        for t in range(NTN):
            if t + 2 < NTN:
                if t - 2 >= 0:
                    stores.pop(t - 2).wait()
                loads[t + 2] = start_load(t + 2)
            loads[t].wait()
            cbase = wid * CW + t * SUBN
            chunk = bufs[t % NB]

            KS = 2

            def kbody(k, _):
                mvs, rvs = [], []
                for u in range(KS):
                    col = cbase + (k * KS + u) * _L + lane_iota
                    seg = jnp.zeros((_L,), jnp.int32)
                    for j in range(1, B):
                        seg = seg + (col >= off_vec[j]).astype(jnp.int32)
                    mvs.append(mean.at[seg].get(mode="promise_in_bounds"))
                    rvs.append(rstd.at[seg].get(mode="promise_in_bounds"))

                def mbody(m):
                    for u in range(KS):
                        d = pl.ds((k * KS + u) * _L, _L)
                        v = chunk[m, d]
                        chunk[m, d] = (v - mvs[u]) * rvs[u]

                plsc.parallel_loop(0, M, unroll=8)(mbody)
                return 0

            lax.fori_loop(0, KVN // KS, kbody, 0)
            stores[t] = start_store(t)

        for t in sorted(stores):
            stores[t].wait()

    return stats_k, norm_k


def kernel(values, offsets, M):
    total, m = values.shape
    B = offsets.shape[0] - 1
    stats_k, norm_k = _build(total, m, B)
    vt = values.T
    part = stats_k(vt, offsets)
    out_t = norm_k(vt, offsets, part)
    return out_t.T


# speedup vs baseline: 1.0342x; 1.0342x over previous
import functools

import jax
import jax.numpy as jnp
from jax import lax
from jax.experimental import pallas as pl
from jax.experimental.pallas import tpu as pltpu
from jax.experimental.pallas import tpu_sc as plsc

_EPS = 1e-6
_L = 16


def _rsqrt_newton(x):
    i = plsc.bitcast(x, jnp.int32)
    i = jnp.full(x.shape, 0x5F3759DF, jnp.int32) - lax.shift_right_logical(i, 1)
    y = plsc.bitcast(i, jnp.float32)
    for _ in range(3):
        y = y * (1.5 - 0.5 * x * y * y)
    return y


@functools.lru_cache(maxsize=None)
def _build(total, M, B):
    mesh = plsc.VectorSubcoreMesh(core_axis_name="c", subcore_axis_name="s")
    NC, NS = mesh.num_cores, mesh.num_subcores
    NW = NC * NS
    CW = total // NW
    SUB = 256
    NT = CW // SUB
    KV = SUB // _L
    SUBN = 256
    NTN = CW // SUBN
    KVN = SUBN // _L
    NB = 3
    assert total == NW * NT * SUB and CW % SUBN == 0

    def seg_cols(off_vec, i):
        lo = off_vec[i]
        hi = jnp.int32(total) if i == B - 1 else off_vec[i + 1]
        return lo, hi

    @functools.partial(
        pl.kernel,
        out_type=jax.ShapeDtypeStruct((NW * 2 * _L,), jnp.float32),
        mesh=mesh,
        compiler_params=pltpu.CompilerParams(needs_layout_passes=False),
        scratch_types=[
            pltpu.VMEM((M, SUB), jnp.float32),
            pltpu.VMEM((M, SUB), jnp.float32),
            pltpu.VMEM((CW,), jnp.float32),
            pltpu.VMEM((CW,), jnp.float32),
            pltpu.VMEM((_L,), jnp.int32),
            pltpu.VMEM((2 * _L,), jnp.float32),
            pltpu.SemaphoreType.DMA,
            pltpu.SemaphoreType.DMA,
        ],
    )
    def stats_k(vt_hbm, offsets_hbm, part_hbm,
                chunk0, chunk1, colsum, colsq, offs, stat_v, sem0, sem1):
        zeros = jnp.zeros((_L,), jnp.float32)
        lane_iota = lax.iota(jnp.int32, _L)
        wid = lax.axis_index("c") * NS + lax.axis_index("s")
        bufs = [chunk0, chunk1]
        sems = [sem0, sem1]

        def start_load(t):
            cb = wid * CW + t * SUB
            return pltpu.async_copy(
                vt_hbm.at[:, pl.ds(cb, SUB)], bufs[t % 2], sems[t % 2])

        loads = {0: start_load(0)}
        pltpu.sync_copy(offsets_hbm.at[pl.ds(0, _L)], offs)
        off_vec = offs[...]
        sums_vec = zeros
        sq_vec = zeros
        for t in range(NT):
            if t + 1 < NT:
                loads[t + 1] = start_load(t + 1)
            loads[t].wait()
            cbase = wid * CW + t * SUB
            chunk = bufs[t % 2]

            def kbody(k, _):
                def mbody(m, carry):
                    s0, q0, s1, q1 = carry
                    v0 = chunk[m, pl.ds(k * 2 * _L, _L)]
                    v1 = chunk[m, pl.ds(k * 2 * _L + _L, _L)]
                    return s0 + v0, q0 + v0 * v0, s1 + v1, q1 + v1 * v1

                s0, q0, s1, q1 = plsc.parallel_loop(
                    0, M, unroll=8,
                    carry=(zeros, zeros, zeros, zeros))(mbody)
                cb = t * SUB + k * 2 * _L
                colsum[pl.ds(cb, _L)] = s0
                colsq[pl.ds(cb, _L)] = q0
                colsum[pl.ds(cb + _L, _L)] = s1
                colsq[pl.ds(cb + _L, _L)] = q1
                return 0

            lax.fori_loop(0, KV // 2, kbody, 0)

        wbase = wid * CW
        for i in range(B):
            lo, hi = seg_cols(off_vec, i)
            ra = jnp.clip(lo - wbase, 0, CW)
            rb = jnp.clip(hi - wbase, 0, CW)

            def sbody(kk, carry):
                s, q = carry
                g = kk * _L + lane_iota
                msk = (g >= ra) & (g < rb)
                s = s + jnp.where(msk, colsum[pl.ds(kk * _L, _L)], 0.0)
                q = q + jnp.where(msk, colsq[pl.ds(kk * _L, _L)], 0.0)
                return s, q

            s, q = lax.fori_loop(
                lax.div(ra, _L), lax.div(rb + (_L - 1), _L),
                sbody, (zeros, zeros))
            lane = lane_iota == i
            sums_vec = jnp.where(lane, sums_vec + jnp.sum(s), sums_vec)
            sq_vec = jnp.where(lane, sq_vec + jnp.sum(q), sq_vec)

        stat_v[pl.ds(0, _L)] = sums_vec
        stat_v[pl.ds(_L, _L)] = sq_vec
        pltpu.sync_copy(stat_v, part_hbm.at[pl.ds(wid * 2 * _L, 2 * _L)])

    @functools.partial(
        pl.kernel,
        out_type=jax.ShapeDtypeStruct((M, total), jnp.float32),
        mesh=mesh,
        compiler_params=pltpu.CompilerParams(needs_layout_passes=False),
        scratch_types=(
            [pltpu.VMEM((M, SUBN), jnp.float32)] * NB
            + [pltpu.VMEM((_L,), jnp.int32),
               pltpu.VMEM((NW * 2 * _L,), jnp.float32)]
            + [pltpu.SemaphoreType.DMA] * (2 * NB)
        ),
    )
    def norm_k(vt_hbm, offsets_hbm, part_hbm, out_hbm, *refs):
        bufs = list(refs[:NB])
        offs, part_v = refs[NB], refs[NB + 1]
        lsems = list(refs[NB + 2:NB + 2 + NB])
        ssems = list(refs[NB + 2 + NB:])
        zeros = jnp.zeros((_L,), jnp.float32)
        lane_iota = lax.iota(jnp.int32, _L)
        wid = lax.axis_index("c") * NS + lax.axis_index("s")

        def start_load(t):
            cb = wid * CW + t * SUBN
            return pltpu.async_copy(
                vt_hbm.at[:, pl.ds(cb, SUBN)], bufs[t % NB], lsems[t % NB])

        def start_store(t):
            cb = wid * CW + t * SUBN
            return pltpu.async_copy(
                bufs[t % NB], out_hbm.at[:, pl.ds(cb, SUBN)], ssems[t % NB])

        loads = {0: start_load(0), 1: start_load(1)}
        pltpu.sync_copy(offsets_hbm.at[pl.ds(0, _L)], offs)
        pltpu.sync_copy(part_hbm, part_v)
        off_vec = offs[...]

        sums = zeros
        sqs = zeros
        for w in range(NW):
            sums = sums + part_v[pl.ds(w * 2 * _L, _L)]
            sqs = sqs + part_v[pl.ds(w * 2 * _L + _L, _L)]

        off_hi = jnp.full((_L,), total, jnp.int32)
        for i in range(B - 1):
            off_hi = jnp.where(lane_iota == i, off_vec[i + 1], off_hi)
        n_elem = (off_hi - off_vec).astype(jnp.float32) * jnp.float32(M)

        mean = sums / n_elem
        var = sqs / n_elem - mean * mean
        rstd = _rsqrt_newton(var + _EPS)

        stores = {}
        for t in range(NTN):
            if t + 2 < NTN:
                if t + 2 - NB >= 0:
                    stores.pop(t + 2 - NB).wait()
                loads[t + 2] = start_load(t + 2)
            loads[t].wait()
            cbase = wid * CW + t * SUBN
            chunk = bufs[t % NB]

            KS = 2

            def kbody(k, _):
                mvs, rvs = [], []
                for u in range(KS):
                    col = cbase + (k * KS + u) * _L + lane_iota
                    seg = jnp.zeros((_L,), jnp.int32)
                    for j in range(1, B):
                        seg = seg + (col >= off_vec[j]).astype(jnp.int32)
                    mvs.append(mean.at[seg].get(mode="promise_in_bounds"))
                    rvs.append(rstd.at[seg].get(mode="promise_in_bounds"))

                def mbody(m):
                    for u in range(KS):
                        d = pl.ds((k * KS + u) * _L, _L)
                        v = chunk[m, d]
                        chunk[m, d] = (v - mvs[u]) * rvs[u]

                plsc.parallel_loop(0, M, unroll=8)(mbody)
                return 0

            lax.fori_loop(0, KVN // KS, kbody, 0)
            stores[t] = start_store(t)

        for t in sorted(stores):
            stores[t].wait()

    return stats_k, norm_k


def kernel(values, offsets, M):
    total, m = values.shape
    B = offsets.shape[0] - 1
    stats_k, norm_k = _build(total, m, B)
    vt = values.T
    part = stats_k(vt, offsets)
    out_t = norm_k(vt, offsets, part)
    return out_t.T
